# baseline (speedup 1.0000x reference)
import jax
import jax.numpy as jnp
from jax.experimental import pallas as pl
from jax.experimental.pallas import tpu as pltpu

_CHUNK = 5000
_NBUF = 2


def _affine_kernel(x_hbm, w_ref, b_ref, o_hbm, xbuf, obuf, ld_sem, st_sem):
    n = x_hbm.shape[0]
    nc = n // _CHUNK
    w = w_ref[...]
    bias = b_ref[...]

    def ld(i, slot):
        return pltpu.make_async_copy(
            x_hbm.at[pl.ds(i * _CHUNK, _CHUNK), :], xbuf.at[slot],
            ld_sem.at[slot])

    def st(i, slot):
        return pltpu.make_async_copy(
            obuf.at[slot], o_hbm.at[pl.ds(i * _CHUNK, _CHUNK), :],
            st_sem.at[slot])

    ld(0, 0).start()
    for i in range(nc):
        slot = i % _NBUF
        ld(i, slot).wait()
        if i + 1 < nc:
            ld(i + 1, (i + 1) % _NBUF).start()
        if i >= _NBUF:
            st(i - _NBUF, slot).wait()
        obuf[slot] = (
            jnp.dot(xbuf[slot], w, preferred_element_type=jnp.float32) + bias
        )
        st(i, slot).start()
    for i in range(max(nc - _NBUF, 0), nc):
        st(i, i % _NBUF).wait()


def kernel(nodes, edges, senders, receivers, W, b):
    n, d = nodes.shape
    b2 = b.reshape(1, d)
    return pl.pallas_call(
        _affine_kernel,
        in_specs=[
            pl.BlockSpec(memory_space=pltpu.MemorySpace.HBM),
            pl.BlockSpec(memory_space=pltpu.VMEM),
            pl.BlockSpec(memory_space=pltpu.VMEM),
        ],
        out_specs=pl.BlockSpec(memory_space=pltpu.MemorySpace.HBM),
        out_shape=jax.ShapeDtypeStruct((n, d), jnp.float32),
        scratch_shapes=[
            pltpu.VMEM((_NBUF, _CHUNK, d), jnp.float32),
            pltpu.VMEM((_NBUF, _CHUNK, d), jnp.float32),
            pltpu.SemaphoreType.DMA((_NBUF,)),
            pltpu.SemaphoreType.DMA((_NBUF,)),
        ],
    )(nodes, W, b2)


# speedup vs baseline: 1.4237x; 1.4237x over previous
import jax
import jax.numpy as jnp
from jax.experimental import pallas as pl
from jax.experimental.pallas import tpu as pltpu

_BLOCK_ROWS = 5000


def _affine_kernel(x_ref, w_ref, b_ref, o_ref):
    o_ref[...] = (
        jnp.dot(x_ref[...], w_ref[...], preferred_element_type=jnp.float32)
        + b_ref[...]
    )


def kernel(nodes, edges, senders, receivers, W, b):
    n, d = nodes.shape
    grid = (n // _BLOCK_ROWS,)
    b2 = b.reshape(1, d)
    one = pl.Buffered(buffer_count=1)
    return pl.pallas_call(
        _affine_kernel,
        grid=grid,
        in_specs=[
            pl.BlockSpec((_BLOCK_ROWS, d), lambda i: (i, 0)),
            pl.BlockSpec((d, d), lambda i: (0, 0), pipeline_mode=one),
            pl.BlockSpec((1, d), lambda i: (0, 0), pipeline_mode=one),
        ],
        out_specs=pl.BlockSpec((_BLOCK_ROWS, d), lambda i: (i, 0)),
        out_shape=jax.ShapeDtypeStruct((n, d), jnp.float32),
        compiler_params=pltpu.CompilerParams(
            dimension_semantics=("arbitrary",),
        ),
    )(nodes, W, b2)
